# baseline (speedup 1.0000x reference)
import functools

import jax
import jax.numpy as jnp
from jax import lax
from jax.experimental import pallas as pl
from jax.experimental.pallas import tpu as pltpu
from jax.experimental.pallas import tpu_sc as plsc

_NUM_CORES = 2
_NUM_SUBCORES = 16
_NUM_WORKERS = _NUM_CORES * _NUM_SUBCORES
_CHUNK = 256
_L = 16


def _emb_lookup(ids2_t, w2, b, h, d):
    span = b // _NUM_WORKERS
    nsub = span // _CHUNK
    niter = h * nsub
    assert niter % 4 == 0 and d == 64 and _CHUNK % 128 == 0
    mesh = plsc.VectorSubcoreMesh(core_axis_name="c", subcore_axis_name="s")

    @functools.partial(
        pl.kernel,
        out_type=jax.ShapeDtypeStruct((h, d // 8, b // 128, 8, 128), jnp.float32),
        mesh=mesh,
        scratch_types=[
            pltpu.VMEM((4, _CHUNK), jnp.int32),
            pltpu.VMEM((2, _CHUNK, d), jnp.float32),
            pltpu.VMEM((2, d // 8, _CHUNK // 128, 8, 129), jnp.float32),
            pltpu.SemaphoreType.DMA((4,)),
            pltpu.SemaphoreType.DMA((2,)),
            pltpu.SemaphoreType.DMA((2,)),
        ],
        compiler_params=pltpu.CompilerParams(
            use_tc_tiling_on_sc=False, needs_layout_passes=False
        ),
    )
    def emb(ids_hbm, w_hbm, out_hbm, idx_v, rows_v, rt_v, isem, gsem, ssem):
        wid = lax.axis_index("s") * _NUM_CORES + lax.axis_index("c")
        base = wid * span

        def ids_slice(it):
            return ids_hbm.at[
                it // nsub, pl.ds(base + (it % nsub) * _CHUNK, _CHUNK)
            ]

        def start_idx(it, s):
            pltpu.async_copy(ids_slice(it), idx_v.at[s], isem.at[s])

        def wait_idx(s):
            pltpu.make_async_copy(ids_slice(0), idx_v.at[s], isem.at[s]).wait()

        def start_gather(s, p):
            pltpu.async_copy(w_hbm.at[idx_v.at[s]], rows_v.at[p], gsem.at[p])

        def wait_gather(p):
            pltpu.make_async_copy(
                w_hbm.at[idx_v.at[0]], rows_v.at[p], gsem.at[p]
            ).wait()

        def out_slice(it):
            tc = base // 128 + (it % nsub) * (_CHUNK // 128)
            return out_hbm.at[it // nsub, :, pl.ds(tc, _CHUNK // 128), :, :]

        def rt_view(p):
            return rt_v.at[p, :, :, :, pl.ds(0, 128)]

        def start_store(it, p):
            pltpu.async_copy(rt_view(p), out_slice(it), ssem.at[p])

        def wait_store(it, p):
            pltpu.make_async_copy(rt_view(p), out_slice(it), ssem.at[p]).wait()

        lanes = jnp.arange(_L, dtype=jnp.int32)
        dhi = lanes // 8
        dlo = lanes % 8

        def transpose(p):
            rows = rows_v.at[p]
            rt = rt_v.at[p]

            @plsc.parallel_loop(0, _CHUNK, unroll=8)
            def _(v):
                cvec = jnp.full((_L,), v // 128, jnp.int32)
                vvec = jnp.full((_L,), v % 128, jnp.int32)
                for g in range(d // _L):
                    vals = rows[v, pl.ds(g * _L, _L)]
                    plsc.store_scatter(rt, [2 * g + dhi, cvec, dlo, vvec], vals)

        for s in range(4):
            start_idx(s, s)
        wait_idx(0)
        start_gather(0, 0)
        wait_idx(1)
        start_gather(1, 1)

        @pl.loop(0, niter, step=4)
        def _(t):
            for p in range(4):
                it = t + p
                bb = p % 2
                wait_gather(bb)

                @pl.when(it >= 2)
                def _():
                    wait_store(it - 2, bb)

                transpose(bb)
                start_store(it, bb)

                @pl.when(it + 2 < niter)
                def _():
                    wait_idx((p + 2) % 4)
                    start_gather((p + 2) % 4, bb)

                @pl.when(it + 4 < niter)
                def _():
                    start_idx(it + 4, p)

        wait_store(niter - 2, 0)
        wait_store(niter - 1, 1)

    return emb(ids2_t, w2)


def kernel(ids, weight):
    b, h = ids.shape
    v, d = weight.shape
    w2 = jnp.pad(weight, ((0, 0), (0, 128 - d))).reshape(2 * v, d)
    ids2_t = jnp.transpose(ids).astype(jnp.int32) * 2
    out5 = _emb_lookup(ids2_t, w2, b, h, d)
    out = out5.transpose(0, 1, 3, 2, 4).reshape(h, d, b)
    return jnp.transpose(out, (2, 0, 1))


# speedup vs baseline: 1.0016x; 1.0016x over previous
import functools

import jax
import jax.numpy as jnp
from jax import lax
from jax.experimental import pallas as pl
from jax.experimental.pallas import tpu as pltpu
from jax.experimental.pallas import tpu_sc as plsc

_NUM_CORES = 2
_NUM_SUBCORES = 16
_NUM_WORKERS = _NUM_CORES * _NUM_SUBCORES
_CHUNK = 256
_L = 16


def _emb_lookup(ids2_t, w2, b, h, d):
    span = b // _NUM_WORKERS
    nsub = span // _CHUNK
    niter = h * nsub
    assert niter % 4 == 0 and d == 64 and _CHUNK % 128 == 0
    mesh = plsc.VectorSubcoreMesh(core_axis_name="c", subcore_axis_name="s")

    @functools.partial(
        pl.kernel,
        out_type=jax.ShapeDtypeStruct((h, d // 8, b // 128, 8, 128), jnp.float32),
        mesh=mesh,
        scratch_types=[
            pltpu.VMEM((4, _CHUNK), jnp.int32),
            pltpu.VMEM((2, _CHUNK, d), jnp.float32),
            pltpu.VMEM((2, d // 8, _CHUNK // 128, 8, 129), jnp.float32),
            pltpu.SemaphoreType.DMA((4,)),
            pltpu.SemaphoreType.DMA((2,)),
            pltpu.SemaphoreType.DMA((2,)),
        ],
        compiler_params=pltpu.CompilerParams(
            use_tc_tiling_on_sc=False, needs_layout_passes=False
        ),
    )
    def emb(ids_hbm, w_hbm, out_hbm, idx_v, rows_v, rt_v, isem, gsem, ssem):
        wid = lax.axis_index("s") * _NUM_CORES + lax.axis_index("c")
        base = wid * span

        def ids_slice(it):
            return ids_hbm.at[
                it // nsub, pl.ds(base + (it % nsub) * _CHUNK, _CHUNK)
            ]

        def start_idx(it, s):
            pltpu.async_copy(ids_slice(it), idx_v.at[s], isem.at[s])

        def wait_idx(s):
            pltpu.make_async_copy(ids_slice(0), idx_v.at[s], isem.at[s]).wait()

        def start_gather(s, p):
            pltpu.async_copy(w_hbm.at[idx_v.at[s]], rows_v.at[p], gsem.at[p])

        def wait_gather(p):
            pltpu.make_async_copy(
                w_hbm.at[idx_v.at[0]], rows_v.at[p], gsem.at[p]
            ).wait()

        def out_slice(it):
            tc = base // 128 + (it % nsub) * (_CHUNK // 128)
            return out_hbm.at[it // nsub, :, pl.ds(tc, _CHUNK // 128), :, :]

        def rt_view(p):
            return rt_v.at[p, :, :, :, pl.ds(0, 128)]

        def start_store(it, p):
            pltpu.async_copy(rt_view(p), out_slice(it), ssem.at[p])

        def wait_store(it, p):
            pltpu.make_async_copy(rt_view(p), out_slice(it), ssem.at[p]).wait()

        lanes = jnp.arange(_L, dtype=jnp.int32)
        dhi = lanes // 8
        dlo = lanes % 8

        def transpose(p):
            rows = rows_v.at[p]
            rt = rt_v.at[p]

            @plsc.parallel_loop(0, _CHUNK, unroll=4)
            def _(v):
                cvec = jnp.full((_L,), v // 128, jnp.int32)
                vvec = jnp.full((_L,), v % 128, jnp.int32)
                for g in range(d // _L):
                    vals = rows[v, pl.ds(g * _L, _L)]
                    plsc.store_scatter(rt, [2 * g + dhi, cvec, dlo, vvec], vals)

        for s in range(4):
            start_idx(s, s)
        wait_idx(0)
        start_gather(0, 0)
        wait_idx(1)
        start_gather(1, 1)

        @pl.loop(0, niter, step=4)
        def _(t):
            for p in range(4):
                it = t + p
                bb = p % 2
                wait_gather(bb)

                @pl.when(it >= 2)
                def _():
                    wait_store(it - 2, bb)

                transpose(bb)
                start_store(it, bb)

                @pl.when(it + 2 < niter)
                def _():
                    wait_idx((p + 2) % 4)
                    start_gather((p + 2) % 4, bb)

                @pl.when(it + 4 < niter)
                def _():
                    start_idx(it + 4, p)

        wait_store(niter - 2, 0)
        wait_store(niter - 1, 1)

    return emb(ids2_t, w2)


def kernel(ids, weight):
    b, h = ids.shape
    v, d = weight.shape
    w2 = jnp.pad(weight, ((0, 0), (0, 128 - d))).reshape(2 * v, d)
    ids2_t = jnp.transpose(ids).astype(jnp.int32) * 2
    out5 = _emb_lookup(ids2_t, w2, b, h, d)
    out = out5.transpose(0, 1, 3, 2, 4).reshape(h, d, b)
    return jnp.transpose(out, (2, 0, 1))
